# Initial kernel scaffold; baseline (speedup 1.0000x reference)
import jax
import jax.numpy as jnp
from jax.experimental import pallas as pl


def kernel(features, num_voxels, conv1_w, bn1_g, bn1_b, conv2_w, bn2_g, bn2_b, conv3_w, bn3_g, bn3_b, conv4_w, bn4_g, bn4_b):
    raise NotImplementedError("write your pallas kernel here")


# speedup vs baseline: 16.0605x; 16.0605x over previous
import jax
import jax.numpy as jnp
from jax.experimental import pallas as pl

_V, _T = 4096, 64
_K = 8
_EPS = 1e-5
_VB = 16
_NB = _V // _VB
_CNT = float(_V * _T)
_BIG = 1e30


def _bdot(a, b):
    return jax.lax.dot_general(
        a, b, (((2,), (2,)), ((0,), (0,))), preferred_element_type=jnp.float32)


def _bmm_hi(a, b):
    return jax.lax.dot_general(
        a, b, (((2,), (1,)), ((0,), (0,))), preferred_element_type=jnp.float32,
        precision=jax.lax.Precision.HIGHEST)


def _dist(x):
    sq = jnp.sum(x * x, axis=2, keepdims=True)
    sqt = jnp.swapaxes(sq, 1, 2)
    xbf = x.astype(jnp.bfloat16)
    return (sq + sqt) - 2.0 * _bdot(xbf, xbf)


def _topk_gather_cat(d, x):
    vb, c = x.shape[0], x.shape[2]
    nd = -d
    iota = jax.lax.broadcasted_iota(jnp.int32, d.shape, 2)
    cols = []
    for _ in range(_K):
        m = jnp.max(nd, axis=2, keepdims=True)
        eq = nd >= m
        sel = jnp.min(jnp.where(eq, iota, _T), axis=2, keepdims=True)
        oh = iota == sel
        g = _bmm_hi(oh.astype(jnp.float32), x)
        cols.append(g.reshape(vb * _T, c))
        nd = jnp.where(oh, -jnp.inf, nd)
    return jnp.concatenate(cols, axis=1)


def _bn(x, p_ref):
    p = p_ref[...]
    return (x - p[0:1]) / p[1:2] * p[2:3] + p[3:4]


def _acc(st_ref, row):
    @pl.when(pl.program_id(0) == 0)
    def _():
        st_ref[...] = jnp.zeros_like(st_ref)
    st_ref[0:1, :] += row[None]


def _acc2(st_ref, a):
    @pl.when(pl.program_id(0) == 0)
    def _():
        st_ref[...] = jnp.zeros_like(st_ref)
    st_ref[0:1, :] += jnp.sum(a, axis=0)[None]
    st_ref[1:2, :] += jnp.sum(a * a, axis=0)[None]


def _p1(f_ref, nv_ref, w1_ref, r1_ref, st_ref, mi_ref):
    f = f_ref[...]
    nv = nv_ref[...].astype(jnp.float32)
    xyz = f[:, :, 0:3]
    mean = jnp.sum(xyz, axis=1, keepdims=True) / nv
    xr = xyz - mean
    g24 = _topk_gather_cat(_dist(xr), xr)
    a1 = jnp.maximum(
        jnp.dot(g24, w1_ref[...], preferred_element_type=jnp.float32), 0.0)
    r1_ref[...] = a1.reshape(_VB, _T, 32)
    _acc(st_ref, jnp.sum(a1, axis=0))
    mi_ref[...] = jnp.max(f[:, :, 3:4], axis=1)


def _pvar(r_ref, m_ref, st_ref):
    c = r_ref.shape[2]
    x = r_ref[...].reshape(_VB * _T, c) - m_ref[...]
    _acc(st_ref, jnp.sum(x * x, axis=0))


def _p2(r1_ref, p1_ref, w2_ref, r2_ref, st_ref):
    x = _bn(r1_ref[...].reshape(_VB * _T, 32), p1_ref)
    a = jnp.maximum(
        jnp.dot(x, w2_ref[...], preferred_element_type=jnp.float32), 0.0)
    r2_ref[...] = a.reshape(_VB, _T, 64)
    _acc(st_ref, jnp.sum(a, axis=0))


def _p3(r2_ref, p2_ref, w3_ref, r3_ref, st_ref):
    x2 = _bn(r2_ref[...].reshape(_VB * _T, 64), p2_ref).reshape(_VB, _T, 64)
    g512 = _topk_gather_cat(_dist(x2), x2)
    a = jnp.maximum(
        jnp.dot(g512, w3_ref[...], preferred_element_type=jnp.float32), 0.0)
    r3_ref[...] = a.reshape(_VB, _T, 96)
    _acc(st_ref, jnp.sum(a, axis=0))


def _p4(r3_ref, nv_ref, p3_ref, w4_ref, mx_ref, mn_ref, st_ref):
    x = _bn(r3_ref[...].reshape(_VB * _T, 96), p3_ref)
    a = jnp.maximum(
        jnp.dot(x, w4_ref[...], preferred_element_type=jnp.float32), 0.0)
    _acc2(st_ref, a)
    a3 = a.reshape(_VB, _T, 128)
    tio = jax.lax.broadcasted_iota(jnp.int32, (_VB, _T, 1), 1)
    vm = (nv_ref[...] > tio).astype(jnp.float32)
    mx_ref[...] = jnp.max(a3 * vm + (vm - 1.0) * _BIG, axis=1)
    mn_ref[...] = jnp.min(a3 * vm + (1.0 - vm) * _BIG, axis=1)


def _p5(mx_ref, mn_ref, mi_ref, nv_ref, p4_ref, out_ref):
    p = p4_ref[...]
    pick = jnp.where(p[2:3] >= 0.0, mx_ref[...], mn_ref[...])
    cand = (pick - p[0:1]) / p[1:2] * p[2:3] + p[3:4]
    haspad = (nv_ref[...] < _T).astype(jnp.float32)
    cand = jnp.maximum(cand, (haspad - 1.0) * _BIG)
    lane = jax.lax.broadcasted_iota(jnp.int32, cand.shape, 1)
    out_ref[...] = jnp.where(lane == 127, mi_ref[...], cand)


def _vspec(c):
    return pl.BlockSpec((_VB, _T, c), lambda i: (i, 0, 0))


def _fixed(shape):
    n = len(shape)
    return pl.BlockSpec(shape, lambda i: (0,) * n)


def _nvspec():
    return pl.BlockSpec((_VB, 1, 1), lambda i: (i, 0, 0))


def _var_pass(r, m, c):
    ss = pl.pallas_call(
        _pvar,
        grid=(_NB,),
        in_specs=[_vspec(c), _fixed((1, c))],
        out_specs=_fixed((8, c)),
        out_shape=jax.ShapeDtypeStruct((8, c), jnp.float32),
    )(r, m.reshape(1, c))
    return ss[0] / _CNT


def _bnp(m, var, g, b):
    return jnp.stack([m, jnp.sqrt(var + _EPS), g, b])


def kernel(features, num_voxels, conv1_w, bn1_g, bn1_b, conv2_w, bn2_g, bn2_b,
           conv3_w, bn3_g, bn3_b, conv4_w, bn4_g, bn4_b):
    f32 = jnp.float32
    nv3 = num_voxels.reshape(_V, 1, 1)
    w1km = jnp.transpose(conv1_w[..., 0], (2, 1, 0)).reshape(_K * 3, 32)

    r1, s1, mi = pl.pallas_call(
        _p1,
        grid=(_NB,),
        in_specs=[_vspec(4), _nvspec(), _fixed((_K * 3, 32))],
        out_specs=[_vspec(32), _fixed((8, 32)),
                   pl.BlockSpec((_VB, 1), lambda i: (i, 0))],
        out_shape=[jax.ShapeDtypeStruct((_V, _T, 32), f32),
                   jax.ShapeDtypeStruct((8, 32), f32),
                   jax.ShapeDtypeStruct((_V, 1), f32)],
    )(features, nv3, w1km)

    m1 = s1[0] / _CNT
    p1 = _bnp(m1, _var_pass(r1, m1, 32), bn1_g, bn1_b)

    r2, s2 = pl.pallas_call(
        _p2,
        grid=(_NB,),
        in_specs=[_vspec(32), _fixed((4, 32)), _fixed((32, 64))],
        out_specs=[_vspec(64), _fixed((8, 64))],
        out_shape=[jax.ShapeDtypeStruct((_V, _T, 64), f32),
                   jax.ShapeDtypeStruct((8, 64), f32)],
    )(r1, p1, conv2_w[:, :, 0].T)

    m2 = s2[0] / _CNT
    p2 = _bnp(m2, _var_pass(r2, m2, 64), bn2_g, bn2_b)
    w3km = jnp.transpose(conv3_w[..., 0], (2, 1, 0)).reshape(_K * 64, 96)

    r3, s3 = pl.pallas_call(
        _p3,
        grid=(_NB,),
        in_specs=[_vspec(64), _fixed((4, 64)), _fixed((_K * 64, 96))],
        out_specs=[_vspec(96), _fixed((8, 96))],
        out_shape=[jax.ShapeDtypeStruct((_V, _T, 96), f32),
                   jax.ShapeDtypeStruct((8, 96), f32)],
    )(r2, p2, w3km)

    m3 = s3[0] / _CNT
    p3 = _bnp(m3, _var_pass(r3, m3, 96), bn3_g, bn3_b)
    w4p = jnp.pad(conv4_w[:, :, 0].T, ((0, 0), (0, 1)))

    mx, mn, st4 = pl.pallas_call(
        _p4,
        grid=(_NB,),
        in_specs=[_vspec(96), _nvspec(), _fixed((4, 96)), _fixed((96, 128))],
        out_specs=[pl.BlockSpec((_VB, 128), lambda i: (i, 0)),
                   pl.BlockSpec((_VB, 128), lambda i: (i, 0)),
                   _fixed((8, 128))],
        out_shape=[jax.ShapeDtypeStruct((_V, 128), f32),
                   jax.ShapeDtypeStruct((_V, 128), f32),
                   jax.ShapeDtypeStruct((8, 128), f32)],
    )(r3, nv3, p3, w4p)

    g4 = jnp.pad(bn4_g, (0, 1), constant_values=1.0)
    b4g = jnp.pad(bn4_b, (0, 1))
    m4 = st4[0] / _CNT
    v4 = jnp.maximum(st4[1] / _CNT - m4 * m4, 0.0)
    p4 = _bnp(m4, v4, g4, b4g)

    eb = 512
    out = pl.pallas_call(
        _p5,
        grid=(_V // eb,),
        in_specs=[pl.BlockSpec((eb, 128), lambda i: (i, 0)),
                  pl.BlockSpec((eb, 128), lambda i: (i, 0)),
                  pl.BlockSpec((eb, 1), lambda i: (i, 0)),
                  pl.BlockSpec((eb, 1), lambda i: (i, 0)),
                  _fixed((4, 128))],
        out_specs=pl.BlockSpec((eb, 128), lambda i: (i, 0)),
        out_shape=jax.ShapeDtypeStruct((_V, 128), f32),
    )(mx, mn, mi, num_voxels.reshape(_V, 1), p4)

    return out
